# Initial kernel scaffold; baseline (speedup 1.0000x reference)
import jax
import jax.numpy as jnp
from jax.experimental import pallas as pl


def kernel(xyz, sparse_xyz, sparse_flow):
    raise NotImplementedError("write your pallas kernel here")


# speedup vs baseline: 37.5427x; 37.5427x over previous
import jax
import jax.numpy as jnp
from jax.experimental import pallas as pl

_BN = 512


def _knn_interp_kernel(q_ref, kx_ref, kf_ref, ksq_ref, out_ref):
    S = kx_ref.shape[0]
    BN = q_ref.shape[1]

    q = q_ref[...]
    qsq = (q[0:1, :] * q[0:1, :]
           + q[1:2, :] * q[1:2, :]
           + q[2:3, :] * q[2:3, :])

    e = jax.lax.dot_general(
        kx_ref[...], q,
        dimension_numbers=(((1,), (0,)), ((), ())),
        preferred_element_type=jnp.float32,
    )
    sqd = (-2.0 * e + qsq) + ksq_ref[...]

    d2e = jnp.zeros((S, BN), jnp.float32)
    for c in range(3):
        diff = kx_ref[:, c : c + 1] - q[c : c + 1, :]
        d2e = d2e + diff * diff

    big = jnp.float32(jnp.inf)

    m1 = jnp.min(sqd, axis=0, keepdims=True)
    c1 = sqd == m1
    sqb = jnp.where(c1, big, sqd)
    m2 = jnp.min(sqb, axis=0, keepdims=True)
    c2 = sqb == m2
    sqc = jnp.where(c2, big, sqb)
    m3 = jnp.min(sqc, axis=0, keepdims=True)
    c3 = sqc == m3

    g1 = jnp.sum(jnp.where(c1, d2e, 0.0), axis=0, keepdims=True)
    g2 = jnp.sum(jnp.where(c2, d2e, 0.0), axis=0, keepdims=True)
    g3 = jnp.sum(jnp.where(c3, d2e, 0.0), axis=0, keepdims=True)

    r1 = 1.0 / jnp.maximum(jnp.sqrt(g1), 1e-10)
    r2 = 1.0 / jnp.maximum(jnp.sqrt(g2), 1e-10)
    r3 = 1.0 / jnp.maximum(jnp.sqrt(g3), 1e-10)
    inorm = 1.0 / (r1 + r2 + r3)

    W = jnp.where(c1, r1, jnp.where(c2, r2, jnp.where(c3, r3, 0.0)))

    outs = []
    for c in range(3):
        fc = kf_ref[:, c : c + 1]
        outs.append(jnp.sum(W * fc, axis=0, keepdims=True) * inorm)
    out_ref[...] = jnp.concatenate(outs, axis=0)


def kernel(xyz, sparse_xyz, sparse_flow):
    B, C, N = xyz.shape
    S = sparse_xyz.shape[2]
    q = xyz[0]
    kx = sparse_xyz[0].T
    kf = sparse_flow[0].T
    ksq = jnp.sum(kx * kx, axis=1, keepdims=True)

    out = pl.pallas_call(
        _knn_interp_kernel,
        grid=(N // _BN,),
        in_specs=[
            pl.BlockSpec((3, _BN), lambda i: (0, i)),
            pl.BlockSpec((S, 3), lambda i: (0, 0)),
            pl.BlockSpec((S, 3), lambda i: (0, 0)),
            pl.BlockSpec((S, 1), lambda i: (0, 0)),
        ],
        out_specs=pl.BlockSpec((3, _BN), lambda i: (0, i)),
        out_shape=jax.ShapeDtypeStruct((3, N), jnp.float32),
    )(q, kx, kf, ksq)
    return out[None]


# speedup vs baseline: 46.1952x; 1.2305x over previous
import jax
import jax.numpy as jnp
from jax.experimental import pallas as pl

_BN = 512


def _knn_interp_kernel(q_ref, kx_ref, kf_ref, ksq_ref, out_ref):
    S = kx_ref.shape[0]
    BN = q_ref.shape[1]

    q = q_ref[...]
    qsq = (q[0:1, :] * q[0:1, :]
           + q[1:2, :] * q[1:2, :]
           + q[2:3, :] * q[2:3, :])

    e = jax.lax.dot_general(
        kx_ref[...], q,
        dimension_numbers=(((1,), (0,)), ((), ())),
        preferred_element_type=jnp.float32,
    )
    sqd = (-2.0 * e + qsq) + ksq_ref[...]

    d2e = jnp.zeros((S, BN), jnp.float32)
    for c in range(3):
        diff = kx_ref[:, c : c + 1] - q[c : c + 1, :]
        d2e = d2e + diff * diff

    big = jnp.float32(jnp.inf)

    m1 = jnp.min(sqd, axis=0, keepdims=True)
    c1 = sqd == m1
    sqb = jnp.where(c1, big, sqd)
    m2 = jnp.min(sqb, axis=0, keepdims=True)
    c2 = sqb == m2
    sqc = jnp.where(c2, big, sqb)
    m3 = jnp.min(sqc, axis=0, keepdims=True)
    c3 = sqc == m3

    union = c1 | c2 | c3
    r = jnp.where(union, jax.lax.rsqrt(jnp.maximum(d2e, 1e-20)), 0.0)

    norm = jnp.sum(r, axis=0, keepdims=True)
    inorm = 1.0 / norm

    outs = []
    for c in range(3):
        fc = kf_ref[:, c : c + 1]
        outs.append(jnp.sum(r * fc, axis=0, keepdims=True) * inorm)
    out_ref[...] = jnp.concatenate(outs, axis=0)


def kernel(xyz, sparse_xyz, sparse_flow):
    B, C, N = xyz.shape
    S = sparse_xyz.shape[2]
    q = xyz[0]
    kx = sparse_xyz[0].T
    kf = sparse_flow[0].T
    ksq = jnp.sum(kx * kx, axis=1, keepdims=True)

    out = pl.pallas_call(
        _knn_interp_kernel,
        grid=(N // _BN,),
        in_specs=[
            pl.BlockSpec((3, _BN), lambda i: (0, i)),
            pl.BlockSpec((S, 3), lambda i: (0, 0)),
            pl.BlockSpec((S, 3), lambda i: (0, 0)),
            pl.BlockSpec((S, 1), lambda i: (0, 0)),
        ],
        out_specs=pl.BlockSpec((3, _BN), lambda i: (0, i)),
        out_shape=jax.ShapeDtypeStruct((3, N), jnp.float32),
    )(q, kx, kf, ksq)
    return out[None]
